# baseline (device time: 106563 ns/iter reference)
import jax
import jax.numpy as jnp
from jax import lax
from jax.experimental import pallas as pl
from jax.experimental.pallas import tpu as pltpu

N_DEV = 32


def kernel(x, w_mat):
    m, k = x.shape
    _, n = w_mat.shape
    chunk_m = m // N_DEV

    def body(x_ref, w_ref, out_ref, acc_ref, send_ref, recv_ref,
             send_sems, recv_sems):
        my = lax.axis_index("i")
        left = lax.rem(my + N_DEV - 1, N_DEV)
        right = lax.rem(my + 1, N_DEV)

        barrier_sem = pltpu.get_barrier_semaphore()
        for nbr in (left, right):
            pl.semaphore_signal(
                barrier_sem, inc=1,
                device_id=(nbr,), device_id_type=pl.DeviceIdType.MESH,
            )
        pl.semaphore_wait(barrier_sem, 2)

        acc_ref[:, :] = jnp.dot(
            x_ref[:, :], w_ref[:, :], preferred_element_type=jnp.float32
        )

        for s in range(N_DEV - 1):
            idx = lax.rem(my - 1 - s + 2 * N_DEV, N_DEV)
            chunk = acc_ref[pl.ds(idx * chunk_m, chunk_m), :]
            if s == 0:
                send_ref[s] = chunk
            else:
                send_ref[s] = chunk + recv_ref[s - 1]
            rdma = pltpu.make_async_remote_copy(
                src_ref=send_ref.at[s],
                dst_ref=recv_ref.at[s],
                send_sem=send_sems.at[s],
                recv_sem=recv_sems.at[s],
                device_id=(right,),
                device_id_type=pl.DeviceIdType.MESH,
            )
            rdma.start()
            rdma.wait()

        out_ref[:, :] = (
            acc_ref[pl.ds(my * chunk_m, chunk_m), :] + recv_ref[N_DEV - 2]
        )

    return pl.pallas_call(
        body,
        out_shape=jax.ShapeDtypeStruct((chunk_m, n), jnp.float32),
        in_specs=[
            pl.BlockSpec(memory_space=pltpu.VMEM),
            pl.BlockSpec(memory_space=pltpu.VMEM),
        ],
        out_specs=pl.BlockSpec(memory_space=pltpu.VMEM),
        scratch_shapes=[
            pltpu.VMEM((m, n), jnp.float32),
            pltpu.VMEM((N_DEV - 1, chunk_m, n), jnp.float32),
            pltpu.VMEM((N_DEV - 1, chunk_m, n), jnp.float32),
            pltpu.SemaphoreType.DMA((N_DEV - 1,)),
            pltpu.SemaphoreType.DMA((N_DEV - 1,)),
        ],
        compiler_params=pltpu.CompilerParams(collective_id=0),
    )(x, w_mat)


# device time: 35335 ns/iter; 3.0158x vs baseline; 3.0158x over previous
import jax
import jax.numpy as jnp
from jax import lax
from jax.experimental import pallas as pl
from jax.experimental.pallas import tpu as pltpu

N_DEV = 32


def kernel(x, w_mat):
    m, k = x.shape
    _, n = w_mat.shape
    chunk_m = m // N_DEV

    def body(x_ref, w_ref, out_ref, acc_ref, send_ref, recv_ref,
             send_sems, recv_sems):
        my = lax.axis_index("i")

        barrier_sem = pltpu.get_barrier_semaphore()
        for j in range(N_DEV - 1):
            nbr = lax.rem(my + 1 + j, N_DEV)
            pl.semaphore_signal(
                barrier_sem, inc=1,
                device_id=(nbr,), device_id_type=pl.DeviceIdType.MESH,
            )
        pl.semaphore_wait(barrier_sem, N_DEV - 1)

        acc_ref[:, :] = jnp.dot(
            x_ref[:, :], w_ref[:, :], preferred_element_type=jnp.float32
        )

        rdmas = []
        for j in range(N_DEV - 1):
            dst = lax.rem(my + 1 + j, N_DEV)
            send_ref[j] = acc_ref[pl.ds(dst * chunk_m, chunk_m), :].astype(
                jnp.bfloat16
            )
            rdma = pltpu.make_async_remote_copy(
                src_ref=send_ref.at[j],
                dst_ref=recv_ref.at[30 - j],
                send_sem=send_sems.at[j],
                recv_sem=recv_sems.at[30 - j],
                device_id=(dst,),
                device_id_type=pl.DeviceIdType.MESH,
            )
            rdma.start()
            rdmas.append(rdma)

        for j in range(N_DEV - 1):
            rdmas[j].wait_recv()
        out_ref[:, :] = (
            acc_ref[pl.ds(my * chunk_m, chunk_m), :]
            + jnp.sum(recv_ref[:, :, :].astype(jnp.float32), axis=0)
        )

        for j in range(N_DEV - 1):
            rdmas[j].wait_send()

    return pl.pallas_call(
        body,
        out_shape=jax.ShapeDtypeStruct((chunk_m, n), jnp.float32),
        in_specs=[
            pl.BlockSpec(memory_space=pltpu.VMEM),
            pl.BlockSpec(memory_space=pltpu.VMEM),
        ],
        out_specs=pl.BlockSpec(memory_space=pltpu.VMEM),
        scratch_shapes=[
            pltpu.VMEM((m, n), jnp.float32),
            pltpu.VMEM((N_DEV - 1, chunk_m, n), jnp.bfloat16),
            pltpu.VMEM((N_DEV - 1, chunk_m, n), jnp.bfloat16),
            pltpu.SemaphoreType.DMA((N_DEV - 1,)),
            pltpu.SemaphoreType.DMA((N_DEV - 1,)),
        ],
        compiler_params=pltpu.CompilerParams(collective_id=0),
    )(x, w_mat)


# device time: 27689 ns/iter; 3.8486x vs baseline; 1.2761x over previous
import jax
import jax.numpy as jnp
from jax import lax
from jax.experimental import pallas as pl
from jax.experimental.pallas import tpu as pltpu

N_DEV = 32
PLANE = 16
S = 4


def kernel(x, w_mat):
    m, k = x.shape
    _, n = w_mat.shape
    chunk_m = m // N_DEV
    ncol = n // S

    def body(x_ref, w_ref, out_ref, acc_ref, xs_ref, xr_ref, ps_ref, pr_ref,
             xs_sem, xr_sem, ps_sem, pr_sem):
        my = lax.axis_index("i")
        b = lax.rem(my, 2)
        yq = lax.rem(my // 2, 4)
        zq = my // 8
        xcoord = b ^ lax.rem(yq, 2)
        prank = zq * 4 + yq
        partner = my ^ 1

        def plane_dest(p2, xc):
            z2 = p2 // 4
            y2 = lax.rem(p2, 4)
            return z2 * 8 + y2 * 2 + (xc ^ lax.rem(y2, 2))

        barrier_sem = pltpu.get_barrier_semaphore()
        pl.semaphore_signal(
            barrier_sem, inc=1,
            device_id=(partner,), device_id_type=pl.DeviceIdType.MESH,
        )
        for dj in range(1, PLANE):
            p2 = lax.rem(prank + dj, PLANE)
            pl.semaphore_signal(
                barrier_sem, inc=1,
                device_id=(plane_dest(p2, xcoord),),
                device_id_type=pl.DeviceIdType.MESH,
            )
        pl.semaphore_wait(barrier_sem, PLANE)

        acc_ref[:, :] = jnp.dot(
            x_ref[:, :], w_ref[:, :], preferred_element_type=jnp.float32
        )

        x_rdmas = []
        for s in range(S):
            cs = slice(s * ncol, (s + 1) * ncol)
            for i in range(PLANE):
                dl = plane_dest(jnp.int32(i), 1 - xcoord)
                xs_ref[s, i * chunk_m:(i + 1) * chunk_m, :] = acc_ref[
                    pl.ds(dl * chunk_m, chunk_m), cs
                ].astype(jnp.bfloat16)
            rdma = pltpu.make_async_remote_copy(
                src_ref=xs_ref.at[s],
                dst_ref=xr_ref.at[s],
                send_sem=xs_sem.at[s],
                recv_sem=xr_sem.at[s],
                device_id=(partner,),
                device_id_type=pl.DeviceIdType.MESH,
            )
            rdma.start()
            x_rdmas.append(rdma)

        p_rdmas = []
        for s in range(S):
            cs = slice(s * ncol, (s + 1) * ncol)
            x_rdmas[s].wait_recv()
            for dj in range(1, PLANE):
                p2 = lax.rem(prank + dj, PLANE)
                dl = plane_dest(p2, xcoord)
                combined = acc_ref[pl.ds(dl * chunk_m, chunk_m), cs] + xr_ref[
                    s, pl.ds(p2 * chunk_m, chunk_m), :
                ].astype(jnp.float32)
                ps_ref[s, dj - 1] = combined.astype(jnp.bfloat16)
                rdma = pltpu.make_async_remote_copy(
                    src_ref=ps_ref.at[s, dj - 1],
                    dst_ref=pr_ref.at[s, dj - 1],
                    send_sem=ps_sem.at[s],
                    recv_sem=pr_sem.at[s],
                    device_id=(dl,),
                    device_id_type=pl.DeviceIdType.MESH,
                )
                rdma.start()
                p_rdmas.append(rdma)

        for s in range(S):
            cs = slice(s * ncol, (s + 1) * ncol)
            for idx in range(PLANE - 1):
                p_rdmas[s * (PLANE - 1) + idx].wait_recv()
            out_ref[:, cs] = (
                acc_ref[pl.ds(my * chunk_m, chunk_m), cs]
                + xr_ref[s, pl.ds(prank * chunk_m, chunk_m), :].astype(
                    jnp.float32
                )
                + jnp.sum(pr_ref[s].astype(jnp.float32), axis=0)
            )

        for r in x_rdmas:
            r.wait_send()
        for r in p_rdmas:
            r.wait_send()

    return pl.pallas_call(
        body,
        out_shape=jax.ShapeDtypeStruct((chunk_m, n), jnp.float32),
        in_specs=[
            pl.BlockSpec(memory_space=pltpu.VMEM),
            pl.BlockSpec(memory_space=pltpu.VMEM),
        ],
        out_specs=pl.BlockSpec(memory_space=pltpu.VMEM),
        scratch_shapes=[
            pltpu.VMEM((m, n), jnp.float32),
            pltpu.VMEM((S, PLANE * chunk_m, ncol), jnp.bfloat16),
            pltpu.VMEM((S, PLANE * chunk_m, ncol), jnp.bfloat16),
            pltpu.VMEM((S, PLANE - 1, chunk_m, ncol), jnp.bfloat16),
            pltpu.VMEM((S, PLANE - 1, chunk_m, ncol), jnp.bfloat16),
            pltpu.SemaphoreType.DMA((S,)),
            pltpu.SemaphoreType.DMA((S,)),
            pltpu.SemaphoreType.DMA((S,)),
            pltpu.SemaphoreType.DMA((S,)),
        ],
        compiler_params=pltpu.CompilerParams(collective_id=0),
    )(x, w_mat)


# device time: 27235 ns/iter; 3.9127x vs baseline; 1.0167x over previous
import jax
import jax.numpy as jnp
from jax import lax
from jax.experimental import pallas as pl
from jax.experimental.pallas import tpu as pltpu

N_DEV = 32
PLANE = 16
S = 4


def kernel(x, w_mat):
    m, k = x.shape
    _, n = w_mat.shape
    chunk_m = m // N_DEV
    half_m = PLANE * chunk_m
    ncol = n // S

    def body(x_ref, w_ref, out_ref, xp_ref, acc_ref, xs_ref, xr_ref, ps_ref,
             pr_ref, xs_sem, xr_sem, ps_sem, pr_sem):
        my = lax.axis_index("i")
        b = lax.rem(my, 2)
        yq = lax.rem(my // 2, 4)
        zq = my // 8
        xcoord = b ^ lax.rem(yq, 2)
        prank = zq * 4 + yq
        partner = my ^ 1

        def plane_dest(p2, xc):
            z2 = p2 // 4
            y2 = lax.rem(p2, 4)
            return z2 * 8 + y2 * 2 + (xc ^ lax.rem(y2, 2))

        barrier_sem = pltpu.get_barrier_semaphore()
        pl.semaphore_signal(
            barrier_sem, inc=1,
            device_id=(partner,), device_id_type=pl.DeviceIdType.MESH,
        )
        for dj in range(1, PLANE):
            p2 = lax.rem(prank + dj, PLANE)
            pl.semaphore_signal(
                barrier_sem, inc=1,
                device_id=(plane_dest(p2, xcoord),),
                device_id_type=pl.DeviceIdType.MESH,
            )
        pl.semaphore_wait(barrier_sem, PLANE)

        for j in range(PLANE):
            p2 = lax.rem(prank + j, PLANE)
            dl_other = plane_dest(p2, 1 - xcoord)
            dl_own = plane_dest(p2, xcoord)
            xp_ref[j * chunk_m:(j + 1) * chunk_m, :] = x_ref[
                pl.ds(dl_other * chunk_m, chunk_m), :
            ].astype(jnp.bfloat16)
            xp_ref[half_m + j * chunk_m:half_m + (j + 1) * chunk_m, :] = x_ref[
                pl.ds(dl_own * chunk_m, chunk_m), :
            ].astype(jnp.bfloat16)

        acc_ref[0:half_m, :] = jnp.dot(
            xp_ref[0:half_m, :], w_ref[:, :].astype(jnp.bfloat16),
            preferred_element_type=jnp.float32,
        )

        x_rdmas = []
        for s in range(S):
            cs = slice(s * ncol, (s + 1) * ncol)
            xs_ref[s] = acc_ref[0:half_m, cs].astype(jnp.bfloat16)
            rdma = pltpu.make_async_remote_copy(
                src_ref=xs_ref.at[s],
                dst_ref=xr_ref.at[s],
                send_sem=xs_sem.at[s],
                recv_sem=xr_sem.at[s],
                device_id=(partner,),
                device_id_type=pl.DeviceIdType.MESH,
            )
            rdma.start()
            x_rdmas.append(rdma)

        acc_ref[half_m:m, :] = jnp.dot(
            xp_ref[half_m:m, :], w_ref[:, :].astype(jnp.bfloat16),
            preferred_element_type=jnp.float32,
        )

        p_rdmas = []
        for s in range(S):
            cs = slice(s * ncol, (s + 1) * ncol)
            x_rdmas[s].wait_recv()
            for dj in range(1, PLANE):
                row = half_m + dj * chunk_m
                combined = acc_ref[row:row + chunk_m, cs] + xr_ref[
                    s, dj * chunk_m:(dj + 1) * chunk_m, :
                ].astype(jnp.float32)
                ps_ref[s, dj - 1] = combined.astype(jnp.bfloat16)
                rdma = pltpu.make_async_remote_copy(
                    src_ref=ps_ref.at[s, dj - 1],
                    dst_ref=pr_ref.at[s, dj - 1],
                    send_sem=ps_sem.at[s],
                    recv_sem=pr_sem.at[s],
                    device_id=(plane_dest(lax.rem(prank + dj, PLANE), xcoord),),
                    device_id_type=pl.DeviceIdType.MESH,
                )
                rdma.start()
                p_rdmas.append(rdma)

        for s in range(S):
            cs = slice(s * ncol, (s + 1) * ncol)
            for idx in range(PLANE - 1):
                p_rdmas[s * (PLANE - 1) + idx].wait_recv()
            out_ref[:, cs] = (
                acc_ref[half_m:half_m + chunk_m, cs]
                + xr_ref[s, 0:chunk_m, :].astype(jnp.float32)
                + jnp.sum(pr_ref[s].astype(jnp.float32), axis=0)
            )

        for r in x_rdmas:
            r.wait_send()
        for r in p_rdmas:
            r.wait_send()

    return pl.pallas_call(
        body,
        out_shape=jax.ShapeDtypeStruct((chunk_m, n), jnp.float32),
        in_specs=[
            pl.BlockSpec(memory_space=pltpu.VMEM),
            pl.BlockSpec(memory_space=pltpu.VMEM),
        ],
        out_specs=pl.BlockSpec(memory_space=pltpu.VMEM),
        scratch_shapes=[
            pltpu.VMEM((m, k), jnp.bfloat16),
            pltpu.VMEM((m, n), jnp.float32),
            pltpu.VMEM((S, half_m, ncol), jnp.bfloat16),
            pltpu.VMEM((S, half_m, ncol), jnp.bfloat16),
            pltpu.VMEM((S, PLANE - 1, chunk_m, ncol), jnp.bfloat16),
            pltpu.VMEM((S, PLANE - 1, chunk_m, ncol), jnp.bfloat16),
            pltpu.SemaphoreType.DMA((S,)),
            pltpu.SemaphoreType.DMA((S,)),
            pltpu.SemaphoreType.DMA((S,)),
            pltpu.SemaphoreType.DMA((S,)),
        ],
        compiler_params=pltpu.CompilerParams(collective_id=0),
    )(x, w_mat)


# device time: 25042 ns/iter; 4.2554x vs baseline; 1.0876x over previous
import os

import jax
import jax.numpy as jnp
from jax import lax
from jax.experimental import pallas as pl
from jax.experimental.pallas import tpu as pltpu

N_DEV = 32
PLANE = 16
S = 8

_PHASES = os.environ.get("KERNEL_PHASES", "all")
_DO_X = _PHASES in ("all", "x")
_DO_PLANE = _PHASES in ("all", "plane")
_DO_BARRIER = os.environ.get("KERNEL_BARRIER", "1") == "1"


def kernel(x, w_mat):
    m, k = x.shape
    _, n = w_mat.shape
    chunk_m = m // N_DEV
    half_m = PLANE * chunk_m
    ncol = n // S

    def body(x_ref, w_ref, out_ref, xp_ref, acc_ref, xs_ref, xr_ref, ps_ref,
             pr_ref, xs_sem, xr_sem, ps_sem, pr_sem, plane_ready):
        my = lax.axis_index("i")
        b = lax.rem(my, 2)
        yq = lax.rem(my // 2, 4)
        zq = my // 8
        xcoord = b ^ lax.rem(yq, 2)
        prank = zq * 4 + yq
        partner = my ^ 1

        def plane_dest(p2, xc):
            z2 = p2 // 4
            y2 = lax.rem(p2, 4)
            return z2 * 8 + y2 * 2 + (xc ^ lax.rem(y2, 2))

        if _DO_BARRIER:
            barrier_sem = pltpu.get_barrier_semaphore()
            pl.semaphore_signal(
                barrier_sem, inc=1,
                device_id=(partner,), device_id_type=pl.DeviceIdType.MESH,
            )
            for dj in range(1, PLANE):
                p2 = lax.rem(prank + dj, PLANE)
                pl.semaphore_signal(
                    plane_ready, inc=1,
                    device_id=(plane_dest(p2, xcoord),),
                    device_id_type=pl.DeviceIdType.MESH,
                )

        for j in range(PLANE):
            p2 = lax.rem(prank + j, PLANE)
            dl_other = plane_dest(p2, 1 - xcoord)
            dl_own = plane_dest(p2, xcoord)
            xp_ref[j * chunk_m:(j + 1) * chunk_m, :] = x_ref[
                pl.ds(dl_other * chunk_m, chunk_m), :
            ].astype(jnp.bfloat16)
            xp_ref[half_m + j * chunk_m:half_m + (j + 1) * chunk_m, :] = x_ref[
                pl.ds(dl_own * chunk_m, chunk_m), :
            ].astype(jnp.bfloat16)

        acc_ref[0:half_m, :] = jnp.dot(
            xp_ref[0:half_m, :], w_ref[:, :].astype(jnp.bfloat16),
            preferred_element_type=jnp.float32,
        )

        if _DO_BARRIER:
            pl.semaphore_wait(barrier_sem, 1)
        x_rdmas = []
        for s in range(S if _DO_X else 0):
            cs = slice(s * ncol, (s + 1) * ncol)
            xs_ref[s] = acc_ref[0:half_m, cs].astype(jnp.bfloat16)
            rdma = pltpu.make_async_remote_copy(
                src_ref=xs_ref.at[s],
                dst_ref=xr_ref.at[s],
                send_sem=xs_sem.at[s],
                recv_sem=xr_sem.at[s],
                device_id=(partner,),
                device_id_type=pl.DeviceIdType.MESH,
            )
            rdma.start()
            x_rdmas.append(rdma)

        acc_ref[half_m:m, :] = jnp.dot(
            xp_ref[half_m:m, :], w_ref[:, :].astype(jnp.bfloat16),
            preferred_element_type=jnp.float32,
        )

        if _DO_BARRIER and _DO_PLANE:
            pl.semaphore_wait(plane_ready, PLANE - 1)
        p_rdmas = []
        for s in range(S if _DO_PLANE else 0):
            cs = slice(s * ncol, (s + 1) * ncol)
            if _DO_X:
                x_rdmas[s].wait_recv()
            for dj in range(1, PLANE):
                row = half_m + dj * chunk_m
                combined = acc_ref[row:row + chunk_m, cs] + xr_ref[
                    s, dj * chunk_m:(dj + 1) * chunk_m, :
                ].astype(jnp.float32)
                ps_ref[s, dj - 1] = combined.astype(jnp.bfloat16)
                rdma = pltpu.make_async_remote_copy(
                    src_ref=ps_ref.at[s, dj - 1],
                    dst_ref=pr_ref.at[s, dj - 1],
                    send_sem=ps_sem.at[s],
                    recv_sem=pr_sem.at[s],
                    device_id=(plane_dest(lax.rem(prank + dj, PLANE), xcoord),),
                    device_id_type=pl.DeviceIdType.MESH,
                )
                rdma.start()
                p_rdmas.append(rdma)

        for s in range(S):
            cs = slice(s * ncol, (s + 1) * ncol)
            if _DO_PLANE:
                for idx in range(PLANE - 1):
                    p_rdmas[s * (PLANE - 1) + idx].wait_recv()
            if _DO_X and not _DO_PLANE:
                x_rdmas[s].wait_recv()
            out_ref[:, cs] = (
                acc_ref[half_m:half_m + chunk_m, cs]
                + xr_ref[s, 0:chunk_m, :].astype(jnp.float32)
                + jnp.sum(pr_ref[s].astype(jnp.float32), axis=0)
            )

        for r in x_rdmas:
            r.wait_send()
        for r in p_rdmas:
            r.wait_send()

    return pl.pallas_call(
        body,
        out_shape=jax.ShapeDtypeStruct((chunk_m, n), jnp.float32),
        in_specs=[
            pl.BlockSpec(memory_space=pltpu.VMEM),
            pl.BlockSpec(memory_space=pltpu.VMEM),
        ],
        out_specs=pl.BlockSpec(memory_space=pltpu.VMEM),
        scratch_shapes=[
            pltpu.VMEM((m, k), jnp.bfloat16),
            pltpu.VMEM((m, n), jnp.float32),
            pltpu.VMEM((S, half_m, ncol), jnp.bfloat16),
            pltpu.VMEM((S, half_m, ncol), jnp.bfloat16),
            pltpu.VMEM((S, PLANE - 1, chunk_m, ncol), jnp.bfloat16),
            pltpu.VMEM((S, PLANE - 1, chunk_m, ncol), jnp.bfloat16),
            pltpu.SemaphoreType.DMA((S,)),
            pltpu.SemaphoreType.DMA((S,)),
            pltpu.SemaphoreType.DMA((S,)),
            pltpu.SemaphoreType.DMA((S,)),
            pltpu.SemaphoreType.REGULAR,
        ],
        compiler_params=pltpu.CompilerParams(
            collective_id=0 if _DO_BARRIER else None
        ),
    )(x, w_mat)
